# baseline (speedup 1.0000x reference)
import jax
import jax.numpy as jnp
from jax.experimental import pallas as pl
from jax.experimental.pallas import tpu as pltpu

B, H, S, D, MAX_S = 8, 16, 16, 128, 4096
BS = 1024


def _body(pos_ref, kv_ref, vv_ref, kc_ref, vc_ref, ko_ref, vo_ref):
    j = pl.program_id(1)
    ko_ref[...] = kc_ref[...]
    vo_ref[...] = vc_ref[...]
    base = j * BS
    for s in range(S):
        local = pos_ref[s] - base
        @pl.when(jnp.logical_and(local >= 0, local < BS))
        def _():
            ko_ref[0, pl.ds(local, 1), :] = kv_ref[0, pl.ds(s, 1), :]
            vo_ref[0, pl.ds(local, 1), :] = vv_ref[0, pl.ds(s, 1), :]


def kernel(input_pos, k_val, v_val, k_cache, v_cache):
    BH = B * H
    kv = k_val.reshape(BH, S, D)
    vv = v_val.reshape(BH, S, D)
    kc = k_cache.reshape(BH, MAX_S, D)
    vc = v_cache.reshape(BH, MAX_S, D)

    grid = (BH, MAX_S // BS)
    val_spec = pl.BlockSpec((1, S, D), lambda i, j: (i, 0, 0))
    cache_spec = pl.BlockSpec((1, BS, D), lambda i, j: (i, j, 0))
    pos_spec = pl.BlockSpec(memory_space=pltpu.SMEM)

    ko, vo = pl.pallas_call(
        _body,
        grid=grid,
        in_specs=[pos_spec, val_spec, val_spec, cache_spec, cache_spec],
        out_specs=[cache_spec, cache_spec],
        out_shape=[
            jax.ShapeDtypeStruct((BH, MAX_S, D), k_cache.dtype),
            jax.ShapeDtypeStruct((BH, MAX_S, D), v_cache.dtype),
        ],
        compiler_params=pltpu.CompilerParams(
            dimension_semantics=("arbitrary", "arbitrary"),
        ),
    )(input_pos, kv, vv, kc, vc)

    return (ko.reshape(B, H, MAX_S, D), vo.reshape(B, H, MAX_S, D))
